# baseline (speedup 1.0000x reference)
import functools

import jax
import jax.numpy as jnp
from jax import lax
from jax.experimental import pallas as pl
from jax.experimental.pallas import tpu as pltpu
from jax.experimental.pallas import tpu_sc as plsc

_L = 16
_C = 8


@jax.jit
def _sc_lookup(inputs_flat, table):
    n = inputs_flat.shape[0]
    mesh = plsc.VectorSubcoreMesh(core_axis_name="c", subcore_axis_name="s")
    nw = mesh.num_cores * mesh.num_subcores
    npw = n // nw
    nck = npw // _C

    @functools.partial(
        pl.kernel,
        out_type=jax.ShapeDtypeStruct((n,), jnp.float32),
        mesh=mesh,
        scratch_types=[
            pltpu.VMEM((npw,), jnp.int32),
            pltpu.VMEM((npw,), jnp.int32),
            pltpu.VMEM((npw,), jnp.float32),
        ] + [pltpu.SemaphoreType.DMA] * (_C + 1),
    )
    def k(idx_hbm, table_hbm, out_hbm, idx_v, rows_v, outf_v, *sems):
        gsems, osem = sems[:_C], sems[_C]
        sid = lax.axis_index("s")
        wid = sid * mesh.num_cores + lax.axis_index("c")
        base = wid * npw

        pltpu.sync_copy(idx_hbm.at[pl.ds(base, npw)], idx_v)

        gcps = [
            pltpu.async_copy(table_hbm.at[idx_v.at[pl.ds(c * nck, nck)]],
                             rows_v.at[pl.ds(c * nck, nck)], gsems[c])
            for c in range(_C)
        ]

        ocps = []
        for c in range(_C):
            gcps[c].wait()

            @pl.loop(c * nck, (c + 1) * nck, step=_L)
            def _(i):
                outf_v[pl.ds(i, _L)] = (
                    rows_v[pl.ds(i, _L)].astype(jnp.float32))

            ocps.append(pltpu.async_copy(
                outf_v.at[pl.ds(c * nck, nck)],
                out_hbm.at[pl.ds(base + c * nck, nck)], osem))

        for cp in ocps:
            cp.wait()

    return k(inputs_flat, table)


def kernel(inputs, table):
    out = _sc_lookup(inputs.reshape(-1), table)
    return out.reshape(inputs.shape)


# speedup vs baseline: 1.0048x; 1.0048x over previous
import functools

import jax
import jax.numpy as jnp
from jax import lax
from jax.experimental import pallas as pl
from jax.experimental.pallas import tpu as pltpu
from jax.experimental.pallas import tpu_sc as plsc

_L = 16
_C = 8


@jax.jit
def _sc_lookup(inputs_flat, table):
    n = inputs_flat.shape[0]
    mesh = plsc.VectorSubcoreMesh(core_axis_name="c", subcore_axis_name="s")
    nw = mesh.num_cores * mesh.num_subcores
    npw = n // nw
    nck = npw // _C

    @functools.partial(
        pl.kernel,
        out_type=jax.ShapeDtypeStruct((n,), jnp.float32),
        mesh=mesh,
        scratch_types=[
            pltpu.VMEM((npw,), jnp.int32),
            pltpu.VMEM((npw,), jnp.int32),
            pltpu.VMEM((npw,), jnp.float32),
        ] + [pltpu.SemaphoreType.DMA] * (_C + 1),
    )
    def k(idx_hbm, table_hbm, out_hbm, idx_v, rows_v, outf_v, *sems):
        gsems, osem = sems[:_C], sems[_C]
        sid = lax.axis_index("s")
        wid = sid * mesh.num_cores + lax.axis_index("c")
        base = wid * npw

        pltpu.sync_copy(idx_hbm.at[pl.ds(base, nck)],
                        idx_v.at[pl.ds(0, nck)])
        gcps = [pltpu.async_copy(table_hbm.at[idx_v.at[pl.ds(0, nck)]],
                                 rows_v.at[pl.ds(0, nck)], gsems[0])]
        pltpu.sync_copy(idx_hbm.at[pl.ds(base + nck, npw - nck)],
                        idx_v.at[pl.ds(nck, npw - nck)])
        gcps += [
            pltpu.async_copy(table_hbm.at[idx_v.at[pl.ds(c * nck, nck)]],
                             rows_v.at[pl.ds(c * nck, nck)], gsems[c])
            for c in range(1, _C)
        ]

        ocps = []
        for c in range(_C):
            gcps[c].wait()

            @pl.loop(c * nck, (c + 1) * nck, step=_L)
            def _(i):
                outf_v[pl.ds(i, _L)] = (
                    rows_v[pl.ds(i, _L)].astype(jnp.float32))

            ocps.append(pltpu.async_copy(
                outf_v.at[pl.ds(c * nck, nck)],
                out_hbm.at[pl.ds(base + c * nck, nck)], osem))

        for cp in ocps:
            cp.wait()

    return k(inputs_flat, table)


def kernel(inputs, table):
    out = _sc_lookup(inputs.reshape(-1), table)
    return out.reshape(inputs.shape)
